# baseline (speedup 1.0000x reference)
import math

import jax
import jax.numpy as jnp
import numpy as np
from jax import lax
from jax.experimental import pallas as pl
from jax.experimental.pallas import tpu as pltpu
from jax.experimental.pallas import tpu_sc as plsc

D_MODEL = 128
SEQ = 200
BATCH = 4096
SCALE = math.sqrt(128.0)
LANES = 16
NW = 32
BCOL = BATCH // NW
NBUF = 4


def _positional_encoding(length, depth):
    half = depth / 2
    positions = np.arange(length)[:, np.newaxis]
    depths = np.arange(half)[np.newaxis, :] / half
    angle_rates = 1 / 1000 ** depths
    angle_rads = positions * angle_rates
    return np.concatenate(
        [np.sin(angle_rads), np.cos(angle_rads)], axis=-1
    ).astype(np.float32)


_PE = _positional_encoding(SEQ, D_MODEL)


def _sc_body(xt_ref, table_ref, pe_hbm, out_ref, pe_v, idx0, idx1, *scratch):
    rows = scratch[0:NBUF]
    g = scratch[NBUF:2 * NBUF]
    st = scratch[2 * NBUF:3 * NBUF]
    idx = (idx0, idx1)

    c = lax.axis_index("c")
    s = lax.axis_index("s")
    wid = s * 2 + c
    b0 = wid * BCOL
    pltpu.sync_copy(pe_hbm, pe_v)

    def load_idx_block(k, blk):
        pltpu.sync_copy(xt_ref.at[pl.ds(NBUF * k, NBUF), pl.ds(b0, BCOL)],
                        idx[blk])

    def start_gather(pos, blk, j):
        pltpu.async_copy(table_ref.at[idx[blk].at[pos]], rows[j], g[j])

    def wait_gather(j):
        pltpu.make_async_copy(table_ref.at[idx0.at[0]], rows[j], g[j]).wait()

    def start_store(r, j):
        pltpu.async_copy(
            rows[j],
            out_ref.at[pl.ds(b0, BCOL), pl.ds(r * D_MODEL, D_MODEL)], st[j])

    def wait_store(r, j):
        pltpu.make_async_copy(
            rows[j],
            out_ref.at[pl.ds(b0, BCOL), pl.ds(r * D_MODEL, D_MODEL)],
            st[j]).wait()

    def compute(r, j):
        rj = rows[j]
        pv = [pe_v[r, pl.ds(v * LANES, LANES)]
              for v in range(D_MODEL // LANES)]

        def row_body(i, carry):
            for v in range(D_MODEL // LANES):
                sl = pl.ds(v * LANES, LANES)
                rj[i, sl] = rj[i, sl] * SCALE + pv[v]
            return carry

        lax.fori_loop(0, BCOL, row_body, 0)

    load_idx_block(0, 0)
    start_gather(0, 0, 0)
    start_gather(1, 0, 1)

    K = SEQ // NBUF
    M = K // 2

    def iter_body(m, carry):
        for kk in range(2):
            k = 2 * m + kk
            blk, nblk = kk, 1 - kk
            for j in range(NBUF):
                r = NBUF * k + j
                if j == 0:
                    if kk == 0:
                        load_idx_block(k + 1, nblk)
                    else:
                        @pl.when(m < M - 1)
                        def _ld():
                            load_idx_block(k + 1, nblk)

                wait_gather(j)
                compute(r, j)
                start_store(r, j)

                jn = (j + 2) % NBUF
                if j < 2:
                    if kk == 0:
                        @pl.when(m > 0)
                        def _w():
                            wait_store(r - 2, jn)
                    else:
                        wait_store(r - 2, jn)
                    start_gather(j + 2, blk, jn)
                else:
                    if kk == 0:
                        wait_store(r - 2, jn)
                        start_gather(j - 2, nblk, jn)
                    else:
                        @pl.when(m < M - 1)
                        def _w2():
                            wait_store(r - 2, jn)
                            start_gather(j - 2, nblk, jn)
        return carry

    lax.fori_loop(0, M, iter_body, 0)

    for j in range(NBUF):
        wait_store(SEQ - NBUF + j, j)


@jax.jit
def _impl(x, table):
    xt = x.T
    mesh = plsc.VectorSubcoreMesh(core_axis_name="c", subcore_axis_name="s")
    scratch = (
        [pltpu.VMEM((SEQ, D_MODEL), jnp.float32)]
        + [pltpu.VMEM((NBUF, BCOL), jnp.int32)] * 2
        + [pltpu.VMEM((BCOL, D_MODEL), jnp.float32)] * NBUF
        + [pltpu.SemaphoreType.DMA] * NBUF
        + [pltpu.SemaphoreType.DMA] * NBUF
    )
    out = pl.kernel(
        _sc_body,
        out_type=jax.ShapeDtypeStruct((BATCH, SEQ * D_MODEL), jnp.float32),
        mesh=mesh,
        scratch_types=scratch,
    )(xt, table, jnp.asarray(_PE))
    return out.reshape(BATCH, SEQ, D_MODEL)


def kernel(x, table):
    return _impl(x, table)


# speedup vs baseline: 1.7548x; 1.7548x over previous
import math

import jax
import jax.numpy as jnp
import numpy as np
from jax import lax
from jax.experimental import pallas as pl
from jax.experimental.pallas import tpu as pltpu
from jax.experimental.pallas import tpu_sc as plsc

D_MODEL = 128
SEQ = 200
BATCH = 4096
SCALE = math.sqrt(128.0)
LANES = 16
NW = 32
NBPW = BATCH // NW
NST = NBPW * 2
NBUF = 8
DIST = NBUF // 2
CH = (104, 96)


def _positional_encoding(length, depth):
    half = depth / 2
    positions = np.arange(length)[:, np.newaxis]
    depths = np.arange(half)[np.newaxis, :] / half
    angle_rates = 1 / 1000 ** depths
    angle_rads = positions * angle_rates
    return np.concatenate(
        [np.sin(angle_rads), np.cos(angle_rads)], axis=-1
    ).astype(np.float32)


_PE = _positional_encoding(SEQ, D_MODEL)


def _sc_body(x_ref, table_ref, pe_hbm, out_ref, pe_v, idx0, idx1, *scratch):
    rows = scratch[0:NBUF]
    g = scratch[NBUF:2 * NBUF]
    st = scratch[2 * NBUF:3 * NBUF]
    idx = (idx0, idx1)

    c = lax.axis_index("c")
    s = lax.axis_index("s")
    wid = s * 2 + c
    batch0 = wid * NBPW
    pltpu.sync_copy(pe_hbm, pe_v)

    def load_idx_block(q, blk):
        pltpu.sync_copy(x_ref.at[pl.ds((batch0 + 2 * q) * SEQ, 2 * SEQ)],
                        idx[blk])

    _POFF = (0, CH[0], SEQ, SEQ + CH[0])

    def start_gather(p, blk, j):
        pltpu.async_copy(
            table_ref.at[idx[blk].at[pl.ds(_POFF[p], CH[j % 2])]],
            rows[j], g[j])

    def wait_gather(j):
        pltpu.make_async_copy(
            table_ref.at[idx0.at[pl.ds(0, CH[j % 2])]], rows[j], g[j]).wait()

    def _out_slice(k, j):
        row0 = (batch0 + 4 * k + j // 2) * SEQ + (j % 2) * CH[0]
        return out_ref.at[pl.ds(row0, CH[j % 2])]

    def start_store(k, j):
        pltpu.async_copy(rows[j], _out_slice(k, j), st[j])

    def wait_store(k, j):
        pltpu.make_async_copy(rows[j], _out_slice(k, j), st[j]).wait()

    def compute(j):
        rj = rows[j]
        pbase = (j % 2) * CH[0]

        def row_body(r, carry):
            for v in range(D_MODEL // LANES):
                sl = pl.ds(v * LANES, LANES)
                rj[r, sl] = rj[r, sl] * SCALE + pe_v[pbase + r, sl]
            return carry

        lax.fori_loop(0, CH[j % 2], row_body, 0)

    load_idx_block(0, 0)
    for j in range(DIST):
        start_gather(j, 0, j)

    K = NST // NBUF

    def iter_body(k, carry):
        for j in range(NBUF):
            if j == 0:
                load_idx_block(2 * k + 1, 1)
            if j == DIST:
                @pl.when(k < K - 1)
                def _ld():
                    load_idx_block(2 * k + 2, 0)

            wait_gather(j)
            compute(j)
            start_store(k, j)

            jn = (j + DIST) % NBUF
            if j < DIST:
                @pl.when(k > 0)
                def _w():
                    wait_store(k - 1, jn)

                start_gather(j, 1, jn)
            else:
                @pl.when(k < K - 1)
                def _w2():
                    wait_store(k, jn)
                    start_gather(j - DIST, 0, jn)
        return carry

    lax.fori_loop(0, K, iter_body, 0)

    for j in range(NBUF):
        wait_store(K - 1, j)


@jax.jit
def _impl(x, table):
    xf = x.reshape(-1)
    mesh = plsc.VectorSubcoreMesh(core_axis_name="c", subcore_axis_name="s")
    scratch = (
        [pltpu.VMEM((SEQ, D_MODEL), jnp.float32)]
        + [pltpu.VMEM((2 * SEQ,), jnp.int32)] * 2
        + [pltpu.VMEM((CH[j % 2], D_MODEL), jnp.float32)
           for j in range(NBUF)]
        + [pltpu.SemaphoreType.DMA] * NBUF
        + [pltpu.SemaphoreType.DMA] * NBUF
    )
    out = pl.kernel(
        _sc_body,
        out_type=jax.ShapeDtypeStruct((BATCH * SEQ, D_MODEL), jnp.float32),
        mesh=mesh,
        scratch_types=scratch,
    )(xf, table, jnp.asarray(_PE))
    return out.reshape(BATCH, SEQ, D_MODEL)


def kernel(x, table):
    return _impl(x, table)


# speedup vs baseline: 1.8346x; 1.0455x over previous
import math

import jax
import jax.numpy as jnp
import numpy as np
from jax import lax
from jax.experimental import pallas as pl
from jax.experimental.pallas import tpu as pltpu
from jax.experimental.pallas import tpu_sc as plsc

D_MODEL = 128
SEQ = 200
BATCH = 4096
SCALE = math.sqrt(128.0)
LANES = 16
NW = 32
NBPW = BATCH // NW
NST = NBPW * 2
NBUF = 8
DIST = NBUF // 2
CH = (104, 96)


def _positional_encoding(length, depth):
    half = depth / 2
    positions = np.arange(length)[:, np.newaxis]
    depths = np.arange(half)[np.newaxis, :] / half
    angle_rates = 1 / 1000 ** depths
    angle_rads = positions * angle_rates
    return np.concatenate(
        [np.sin(angle_rads), np.cos(angle_rads)], axis=-1
    ).astype(np.float32)


_PE = _positional_encoding(SEQ, D_MODEL)


def _sc_body(x_ref, table_ref, pe_hbm, out_ref, pe_v, idx0, idx1, *scratch):
    rows = scratch[0:NBUF]
    g = scratch[NBUF:2 * NBUF]
    st = scratch[2 * NBUF:3 * NBUF]
    idx = (idx0, idx1)

    c = lax.axis_index("c")
    s = lax.axis_index("s")
    wid = s * 2 + c
    batch0 = wid * NBPW
    pltpu.sync_copy(pe_hbm, pe_v)

    def load_idx_block(q, blk):
        pltpu.sync_copy(x_ref.at[pl.ds((batch0 + 2 * q) * SEQ, 2 * SEQ)],
                        idx[blk])

    _POFF = (0, CH[0], SEQ, SEQ + CH[0])

    def start_gather(p, blk, j):
        pltpu.async_copy(
            table_ref.at[idx[blk].at[pl.ds(_POFF[p], CH[j % 2])]],
            rows[j], g[j])

    def wait_gather(j):
        pltpu.make_async_copy(
            table_ref.at[idx0.at[pl.ds(0, CH[j % 2])]], rows[j], g[j]).wait()

    def _out_slice(k, j):
        row0 = (batch0 + 4 * k + j // 2) * SEQ + (j % 2) * CH[0]
        return out_ref.at[pl.ds(row0, CH[j % 2])]

    def start_store(k, j):
        pltpu.async_copy(rows[j], _out_slice(k, j), st[j])

    def wait_store(k, j):
        pltpu.make_async_copy(rows[j], _out_slice(k, j), st[j]).wait()

    def compute_pair(ja, jb):
        ra, rb = rows[ja], rows[jb]
        pbase = (ja % 2) * CH[0]

        def row_body(r, carry):
            for v in range(D_MODEL // LANES):
                sl = pl.ds(v * LANES, LANES)
                pv = pe_v[pbase + r, sl]
                ra[r, sl] = ra[r, sl] * SCALE + pv
                rb[r, sl] = rb[r, sl] * SCALE + pv
            return carry

        lax.fori_loop(0, CH[ja % 2], row_body, 0)

    load_idx_block(0, 0)
    for j in range(DIST):
        start_gather(j, 0, j)

    K = NST // NBUF

    def iter_body(k, carry):
        for j in range(NBUF):
            if j == 0:
                load_idx_block(2 * k + 1, 1)
            if j == DIST:
                @pl.when(k < K - 1)
                def _ld():
                    load_idx_block(2 * k + 2, 0)

            if j in (2, 3):
                wait_gather(j - 2)
                wait_gather(j)
                compute_pair(j - 2, j)
                start_store(k, j - 2)
                start_store(k, j)
            elif j in (6, 7):
                wait_gather(j - 2)
                wait_gather(j)
                compute_pair(j - 2, j)
                start_store(k, j - 2)
                start_store(k, j)

            jn = (j + DIST) % NBUF
            if j < DIST:
                @pl.when(k > 0)
                def _w():
                    wait_store(k - 1, jn)

                start_gather(j, 1, jn)
            else:
                @pl.when(k < K - 1)
                def _w2():
                    wait_store(k, jn)
                    start_gather(j - DIST, 0, jn)
        return carry

    lax.fori_loop(0, K, iter_body, 0)

    for j in range(NBUF):
        wait_store(K - 1, j)


@jax.jit
def _impl(x, table):
    xf = x.reshape(-1)
    mesh = plsc.VectorSubcoreMesh(core_axis_name="c", subcore_axis_name="s")
    scratch = (
        [pltpu.VMEM((SEQ, D_MODEL), jnp.float32)]
        + [pltpu.VMEM((2 * SEQ,), jnp.int32)] * 2
        + [pltpu.VMEM((CH[j % 2], D_MODEL), jnp.float32)
           for j in range(NBUF)]
        + [pltpu.SemaphoreType.DMA] * NBUF
        + [pltpu.SemaphoreType.DMA] * NBUF
    )
    out = pl.kernel(
        _sc_body,
        out_type=jax.ShapeDtypeStruct((BATCH * SEQ, D_MODEL), jnp.float32),
        mesh=mesh,
        scratch_types=scratch,
    )(xf, table, jnp.asarray(_PE))
    return out.reshape(BATCH, SEQ, D_MODEL)


def kernel(x, table):
    return _impl(x, table)
